# initial kernel scaffold (unmeasured)
import functools

import jax
import jax.numpy as jnp
from jax import lax
from jax.experimental import pallas as pl
from jax.experimental.pallas import tpu as pltpu

N_DEV = 4
M_SH = 2048
K = 8192
N_SH = 1024
KB = 2048
NK = K // KB


def _body(sched_ref, x_ref, w_ref, out_ref,
          acc_ref, send_bufs, recv_bufs, send_sems, recv_sems):
    j = pl.program_id(0)
    k = pl.program_id(1)
    my_i = lax.axis_index("i")

    @pl.when((j == 0) & (k == 0))
    def _barrier():
        bsem = pltpu.get_barrier_semaphore()
        for h in (1, 2, 3):
            pl.semaphore_signal(
                bsem, inc=1,
                device_id=((my_i + h) % N_DEV,),
                device_id_type=pl.DeviceIdType.MESH,
            )
        pl.semaphore_wait(bsem, N_DEV - 1)

    @pl.when(k == 0)
    def _init():
        acc_ref[...] = jnp.zeros((M_SH, N_SH), jnp.float32)

    acc_ref[...] += lax.dot_general(
        x_ref[:, pl.ds(k * KB, KB)], w_ref[...],
        (((1,), (0,)), ((), ())),
        preferred_element_type=jnp.float32,
    )

    for jj in range(N_DEV - 1):
        @pl.when((j == jj) & (k == NK - 1))
        def _send(jj=jj):
            send_bufs[jj, :, :] = acc_ref[...].astype(jnp.bfloat16)
            rdma = pltpu.make_async_remote_copy(
                src_ref=send_bufs.at[jj],
                dst_ref=recv_bufs.at[jj],
                send_sem=send_sems.at[jj],
                recv_sem=recv_sems.at[jj],
                device_id=((my_i + 1 + jj) % N_DEV,),
                device_id_type=pl.DeviceIdType.MESH,
            )
            rdma.start()

    @pl.when((j == N_DEV - 1) & (k == NK - 1))
    def _finish():
        out_ref[pl.ds(my_i * M_SH, M_SH), :] = acc_ref[...]
        for jj in range(N_DEV - 1):
            rdma = pltpu.make_async_remote_copy(
                src_ref=send_bufs.at[jj],
                dst_ref=recv_bufs.at[jj],
                send_sem=send_sems.at[jj],
                recv_sem=recv_sems.at[jj],
                device_id=((my_i + 1 + jj) % N_DEV,),
                device_id_type=pl.DeviceIdType.MESH,
            )
            rdma.wait_send()
            rdma.wait_recv()
            src = (my_i - (jj + 1)) % N_DEV
            out_ref[pl.ds(src * M_SH, M_SH), :] = (
                recv_bufs[jj, :, :].astype(jnp.float32))


def kernel(x, w_mat):
    my_i = lax.axis_index("i")
    sched = (my_i + 1 + jnp.arange(N_DEV, dtype=jnp.int32)) % N_DEV

    grid_spec = pltpu.PrefetchScalarGridSpec(
        num_scalar_prefetch=1,
        grid=(N_DEV, NK),
        in_specs=[
            pl.BlockSpec((M_SH, K), lambda j, k, sched: (0, 0)),
            pl.BlockSpec((KB, N_SH), lambda j, k, sched: (k, sched[j])),
        ],
        out_specs=pl.BlockSpec((N_DEV * M_SH, N_SH), lambda j, k, sched: (0, 0)),
        scratch_shapes=[
            pltpu.VMEM((M_SH, N_SH), jnp.float32),
            pltpu.VMEM((N_DEV - 1, M_SH, N_SH), jnp.bfloat16),
            pltpu.VMEM((N_DEV - 1, M_SH, N_SH), jnp.bfloat16),
            pltpu.SemaphoreType.DMA((N_DEV - 1,)),
            pltpu.SemaphoreType.DMA((N_DEV - 1,)),
        ],
    )
    return pl.pallas_call(
        _body,
        out_shape=jax.ShapeDtypeStruct((N_DEV * M_SH, N_SH), jnp.float32),
        grid_spec=grid_spec,
        compiler_params=pltpu.CompilerParams(
            dimension_semantics=("arbitrary", "arbitrary"),
            collective_id=0,
        ),
    )(sched, x, w_mat)


# baseline (device time: 207987 ns/iter reference)
import jax
import jax.numpy as jnp
from jax import lax
from jax.experimental import pallas as pl
from jax.experimental.pallas import tpu as pltpu

N_DEV = 4
M_SH = 2048
K = 8192
N_SH = 1024
KB = 1024
NK = K // KB


def _body(sched_ref, x_ref, w_ref, out_ref,
          acc_ref, send_bufs, recv_bufs, send_sems, recv_sems, out_sem):
    j = pl.program_id(0)
    k = pl.program_id(1)
    my_i = lax.axis_index("i")

    @pl.when((j == 0) & (k == 0))
    def _barrier():
        bsem = pltpu.get_barrier_semaphore()
        for h in (1, 2, 3):
            pl.semaphore_signal(
                bsem, inc=1,
                device_id=((my_i + h) % N_DEV,),
                device_id_type=pl.DeviceIdType.MESH,
            )
        pl.semaphore_wait(bsem, N_DEV - 1)

    @pl.when(k == 0)
    def _init():
        acc_ref[...] = jnp.zeros((M_SH, N_SH), jnp.float32)

    acc_ref[...] += lax.dot_general(
        x_ref[...].astype(jnp.bfloat16), w_ref[...].astype(jnp.bfloat16),
        (((1,), (0,)), ((), ())),
        preferred_element_type=jnp.float32,
    )

    def _rdma(jj):
        return pltpu.make_async_remote_copy(
            src_ref=send_bufs.at[jj],
            dst_ref=recv_bufs.at[jj],
            send_sem=send_sems.at[jj],
            recv_sem=recv_sems.at[jj],
            device_id=((my_i + 1 + jj) % N_DEV,),
            device_id_type=pl.DeviceIdType.MESH,
        )

    for jj in range(N_DEV - 1):
        @pl.when((j == jj) & (k == NK - 1))
        def _send(jj=jj):
            send_bufs[jj, :, :] = acc_ref[...].astype(jnp.bfloat16)
            _rdma(jj).start()

    @pl.when((j == N_DEV - 1) & (k == NK - 1))
    def _finish():
        own = pltpu.make_async_copy(
            acc_ref, out_ref.at[pl.ds(my_i * M_SH, M_SH), :], out_sem)
        own.start()
        own.wait()
        for jj in range(N_DEV - 1):
            rdma = _rdma(jj)
            rdma.wait_recv()
            acc_ref[...] = recv_bufs[jj, :, :].astype(jnp.float32)
            src = (my_i - (jj + 1)) % N_DEV
            cp = pltpu.make_async_copy(
                acc_ref, out_ref.at[pl.ds(src * M_SH, M_SH), :], out_sem)
            cp.start()
            cp.wait()
        for jj in range(N_DEV - 1):
            _rdma(jj).wait_send()


def kernel(x, w_mat):
    my_i = lax.axis_index("i")
    sched = (my_i + 1 + jnp.arange(N_DEV, dtype=jnp.int32)) % N_DEV

    grid_spec = pltpu.PrefetchScalarGridSpec(
        num_scalar_prefetch=1,
        grid=(N_DEV, NK),
        in_specs=[
            pl.BlockSpec((M_SH, KB), lambda j, k, sched: (0, k)),
            pl.BlockSpec((KB, N_SH), lambda j, k, sched: (k, sched[j])),
        ],
        out_specs=pl.BlockSpec(memory_space=pl.ANY),
        scratch_shapes=[
            pltpu.VMEM((M_SH, N_SH), jnp.float32),
            pltpu.VMEM((N_DEV - 1, M_SH, N_SH), jnp.bfloat16),
            pltpu.VMEM((N_DEV - 1, M_SH, N_SH), jnp.bfloat16),
            pltpu.SemaphoreType.DMA((N_DEV - 1,)),
            pltpu.SemaphoreType.DMA((N_DEV - 1,)),
            pltpu.SemaphoreType.DMA,
        ],
    )
    return pl.pallas_call(
        _body,
        out_shape=jax.ShapeDtypeStruct((N_DEV * M_SH, N_SH), jnp.float32),
        grid_spec=grid_spec,
        compiler_params=pltpu.CompilerParams(
            dimension_semantics=("arbitrary", "arbitrary"),
            collective_id=0,
            vmem_limit_bytes=67_000_000,
        ),
    )(sched, x, w_mat)


# device time: 203390 ns/iter; 1.0226x vs baseline; 1.0226x over previous
import os

import jax
import jax.numpy as jnp
from jax import lax
from jax.experimental import pallas as pl
from jax.experimental.pallas import tpu as pltpu

_COMPUTE_ONLY = os.environ.get("K_COMPUTE_ONLY") == "1"

N_DEV = 4
M_SH = 2048
K = 8192
N_SH = 1024
KB = 1024
NK = K // KB


def _body(sched_ref, x_ref, w_ref, out_ref,
          acc_ref, send_bufs, recv_bufs, send_sems, recv_sems, out_sem):
    j = pl.program_id(0)
    k = pl.program_id(1)
    my_i = lax.axis_index("i")

    @pl.when((j == 0) & (k == 0))
    def _barrier():
        bsem = pltpu.get_barrier_semaphore()
        for h in (1, 2, 3):
            pl.semaphore_signal(
                bsem, inc=1,
                device_id=((my_i + h) % N_DEV,),
                device_id_type=pl.DeviceIdType.MESH,
            )
        pl.semaphore_wait(bsem, N_DEV - 1)

    @pl.when(k == 0)
    def _init():
        acc_ref[...] = jnp.zeros((M_SH, N_SH), jnp.float32)

    acc_ref[...] += lax.dot_general(
        x_ref[...].astype(jnp.bfloat16), w_ref[...].astype(jnp.bfloat16),
        (((1,), (0,)), ((), ())),
        preferred_element_type=jnp.float32,
    )

    def _rdma(jj):
        return pltpu.make_async_remote_copy(
            src_ref=send_bufs.at[jj],
            dst_ref=recv_bufs.at[jj],
            send_sem=send_sems.at[jj],
            recv_sem=recv_sems.at[jj],
            device_id=((my_i + 1 + jj) % N_DEV,),
            device_id_type=pl.DeviceIdType.MESH,
        )

    if _COMPUTE_ONLY:
        @pl.when(k == NK - 1)
        def _store_local():
            own = pltpu.make_async_copy(
                acc_ref, out_ref.at[pl.ds(my_i * M_SH, M_SH), :], out_sem)
            own.start()
            own.wait()
        return

    for jj in range(N_DEV - 1):
        @pl.when((j == jj) & (k == NK - 1))
        def _send(jj=jj):
            send_bufs[jj, :, :] = acc_ref[...].astype(jnp.bfloat16)
            _rdma(jj).start()

    @pl.when((j == N_DEV - 1) & (k == NK - 1))
    def _finish():
        own = pltpu.make_async_copy(
            acc_ref, out_ref.at[pl.ds(my_i * M_SH, M_SH), :], out_sem)
        own.start()
        own.wait()
        for jj in range(N_DEV - 1):
            rdma = _rdma(jj)
            rdma.wait_recv()
            acc_ref[...] = recv_bufs[jj, :, :].astype(jnp.float32)
            src = (my_i - (jj + 1)) % N_DEV
            cp = pltpu.make_async_copy(
                acc_ref, out_ref.at[pl.ds(src * M_SH, M_SH), :], out_sem)
            cp.start()
            cp.wait()
        for jj in range(N_DEV - 1):
            _rdma(jj).wait_send()


def kernel(x, w_mat):
    my_i = lax.axis_index("i")
    sched = (my_i + 1 + jnp.arange(N_DEV, dtype=jnp.int32)) % N_DEV

    grid_spec = pltpu.PrefetchScalarGridSpec(
        num_scalar_prefetch=1,
        grid=(N_DEV, NK),
        in_specs=[
            pl.BlockSpec((M_SH, KB), lambda j, k, sched: (0, k)),
            pl.BlockSpec((KB, N_SH), lambda j, k, sched: (k, sched[j])),
        ],
        out_specs=pl.BlockSpec(memory_space=pl.ANY),
        scratch_shapes=[
            pltpu.VMEM((M_SH, N_SH), jnp.float32),
            pltpu.VMEM((N_DEV - 1, M_SH, N_SH), jnp.bfloat16),
            pltpu.VMEM((N_DEV - 1, M_SH, N_SH), jnp.bfloat16),
            pltpu.SemaphoreType.DMA((N_DEV - 1,)),
            pltpu.SemaphoreType.DMA((N_DEV - 1,)),
            pltpu.SemaphoreType.DMA,
        ],
    )
    return pl.pallas_call(
        _body,
        out_shape=jax.ShapeDtypeStruct((N_DEV * M_SH, N_SH), jnp.float32),
        grid_spec=grid_spec,
        compiler_params=pltpu.CompilerParams(
            dimension_semantics=("arbitrary", "arbitrary"),
            collective_id=0,
            vmem_limit_bytes=67_000_000,
        ),
    )(sched, x, w_mat)
